# Initial kernel scaffold; baseline (speedup 1.0000x reference)
import jax
import jax.numpy as jnp
from jax.experimental import pallas as pl


def kernel(xx, ss, lin_in_W, lin_in_b, conv0_Wl, conv0_bl, conv0_Wr, conv1_Wl, conv1_bl, conv1_Wr, norm_g, norm_b, act_lin_W, act_lin_b, scalar_W, scalar_b, tf0_ln1_g, tf0_ln1_b, tf0_Wq, tf0_bq, tf0_Wk, tf0_bk, tf0_Wv, tf0_bv, tf0_Wo, tf0_bo, tf0_ln2_g, tf0_ln2_b, tf0_W1, tf0_b1, tf0_W2, tf0_b2, tf1_ln1_g, tf1_ln1_b, tf1_Wq, tf1_bq, tf1_Wk, tf1_bk, tf1_Wv, tf1_bv, tf1_Wo, tf1_bo, tf1_ln2_g, tf1_ln2_b, tf1_W1, tf1_b1, tf1_W2, tf1_b2):
    raise NotImplementedError("write your pallas kernel here")


# speedup vs baseline: 601.3608x; 601.3608x over previous
import functools

import jax
import jax.numpy as jnp
from jax.experimental import pallas as pl

B, T, S, C = 8, 8, 8, 64
S3 = S ** 3
NH, DH = 4, 16
N_TF = 2
GNN_LAYERS = 2


def _layernorm(x, g, b, eps=1e-5):
    mu = jnp.mean(x, axis=-1, keepdims=True)
    var = jnp.mean((x - mu) ** 2, axis=-1, keepdims=True)
    return (x - mu) * jax.lax.rsqrt(var + eps) * g + b


def _body(t0_ref, acts_ref, ss_ref, *refs):
    out_ref = refs[-1]
    vals = [r[...] for r in refs[:-1]]
    (lin_in_W, lin_in_b,
     conv0_Wl, conv0_bl, conv0_Wr,
     conv1_Wl, conv1_bl, conv1_Wr,
     norm_g, norm_b,
     act_lin_W, act_lin_b,
     scalar_W, scalar_b) = vals[:14]
    tf = vals[14:]

    t0 = t0_ref[0]
    acts = acts_ref[0]
    ssb = ss_ref[0, 0, 0]

    rowidx = jax.lax.broadcasted_iota(jnp.int32, (S3, 1), 0)
    maskf_raw = (t0 != 0.0).astype(jnp.float32)
    anym = jnp.max(maskf_raw)
    maskf = jnp.where(jnp.logical_and(anym == 0.0, rowidx == 0), 1.0, maskf_raw)

    inv_s1 = 1.0 / (S - 1)
    ii = (rowidx // (S * S)).astype(jnp.float32) * inv_s1
    jj = ((rowidx // S) % S).astype(jnp.float32) * inv_s1
    kk = (rowidx % S).astype(jnp.float32) * inv_s1

    x = (ii * lin_in_W[0:1, :] + jj * lin_in_W[1:2, :] + kk * lin_in_W[2:3, :]
         + (t0 * 0.5) * lin_in_W[3:4, :] + (ssb / S) * lin_in_W[4:5, :]
         + lin_in_b)

    m = jnp.sum(maskf)
    deg = jnp.maximum(maskf * (m - maskf), 1.0)
    inv_deg = 1.0 / deg
    for Wl, bl, Wr in ((conv0_Wl, conv0_bl, conv0_Wr),
                       (conv1_Wl, conv1_bl, conv1_Wr)):
        Sx = jnp.sum(maskf * x, axis=0, keepdims=True)
        agg = maskf * (Sx - x) * inv_deg
        h = (jnp.dot(agg, Wl, preferred_element_type=jnp.float32) + bl
             + jnp.dot(x, Wr, preferred_element_type=jnp.float32))
        x = _layernorm(jnp.maximum(h, 0.0), norm_g, norm_b)

    full = x * maskf

    r2 = jax.lax.broadcasted_iota(jnp.int32, (24, S3), 0)
    c2 = jax.lax.broadcasted_iota(jnp.int32, (24, S3), 1)
    ii2 = c2 // (S * S)
    jj2 = (c2 // S) % S
    kk2 = c2 % S
    sel = jnp.where(r2 < 8, ii2, jnp.where(r2 < 16, jj2, kk2))
    poolW = (sel == (r2 % 8)).astype(jnp.float32)
    gnn = jnp.dot(poolW, full, preferred_element_type=jnp.float32) * (1.0 / 64.0)

    a = jnp.dot(acts, act_lin_W, preferred_element_type=jnp.float32) + act_lin_b
    inv_sqrt_dh = 1.0 / 4.0
    for l in range(N_TF):
        (ln1_g, ln1_b, Wq, bq, Wk, bk, Wv, bv, Wo, bo,
         ln2_g, ln2_b, W1, b1, W2, b2) = tf[l * 16:(l + 1) * 16]
        xa = _layernorm(a, ln1_g, ln1_b)
        q = jnp.dot(xa, Wq, preferred_element_type=jnp.float32) + bq
        k = jnp.dot(xa, Wk, preferred_element_type=jnp.float32) + bk
        v = jnp.dot(xa, Wv, preferred_element_type=jnp.float32) + bv
        outs = []
        for hh in range(NH):
            qh = q[:, hh * DH:(hh + 1) * DH]
            kh = k[:, hh * DH:(hh + 1) * DH]
            vh = v[:, hh * DH:(hh + 1) * DH]
            sc = jax.lax.dot_general(
                qh, kh, (((1,), (1,)), ((), ())),
                preferred_element_type=jnp.float32) * inv_sqrt_dh
            sc = sc - jnp.max(sc, axis=-1, keepdims=True)
            e = jnp.exp(sc)
            att = e / jnp.sum(e, axis=-1, keepdims=True)
            outs.append(jnp.dot(att, vh, preferred_element_type=jnp.float32))
        o = jnp.concatenate(outs, axis=1)
        a = a + jnp.dot(o, Wo, preferred_element_type=jnp.float32) + bo
        h2 = _layernorm(a, ln2_g, ln2_b)
        ff = jnp.maximum(
            jnp.dot(h2, W1, preferred_element_type=jnp.float32) + b1, 0.0)
        a = a + jnp.dot(ff, W2, preferred_element_type=jnp.float32) + b2

    act_emb = jnp.mean(a, axis=0, keepdims=True)
    mv_emb = jnp.maximum(ssb * scalar_W + scalar_b, 0.0)

    out_ref[0, 0:24, :] = gnn
    out_ref[0, 24:25, :] = act_emb
    out_ref[0, 25:26, :] = mv_emb


def _const_map(ndim):
    return lambda b: (0,) * ndim


def kernel(xx, ss, lin_in_W, lin_in_b, conv0_Wl, conv0_bl, conv0_Wr,
           conv1_Wl, conv1_bl, conv1_Wr, norm_g, norm_b,
           act_lin_W, act_lin_b, scalar_W, scalar_b,
           tf0_ln1_g, tf0_ln1_b, tf0_Wq, tf0_bq, tf0_Wk, tf0_bk,
           tf0_Wv, tf0_bv, tf0_Wo, tf0_bo, tf0_ln2_g, tf0_ln2_b,
           tf0_W1, tf0_b1, tf0_W2, tf0_b2,
           tf1_ln1_g, tf1_ln1_b, tf1_Wq, tf1_bq, tf1_Wk, tf1_bk,
           tf1_Wv, tf1_bv, tf1_Wo, tf1_bo, tf1_ln2_g, tf1_ln2_b,
           tf1_W1, tf1_b1, tf1_W2, tf1_b2):
    t0col = xx[:, 0].reshape(B, S3, 1).astype(jnp.float32)
    actsf = xx[:, 1:].reshape(B, T - 1, S3).astype(jnp.float32)

    def r2d(v):
        return v.reshape(1, -1)

    weights = [
        lin_in_W, r2d(lin_in_b),
        conv0_Wl, r2d(conv0_bl), conv0_Wr,
        conv1_Wl, r2d(conv1_bl), conv1_Wr,
        r2d(norm_g), r2d(norm_b),
        act_lin_W, r2d(act_lin_b),
        scalar_W, r2d(scalar_b),
        r2d(tf0_ln1_g), r2d(tf0_ln1_b), tf0_Wq, r2d(tf0_bq), tf0_Wk, r2d(tf0_bk),
        tf0_Wv, r2d(tf0_bv), tf0_Wo, r2d(tf0_bo), r2d(tf0_ln2_g), r2d(tf0_ln2_b),
        tf0_W1, r2d(tf0_b1), tf0_W2, r2d(tf0_b2),
        r2d(tf1_ln1_g), r2d(tf1_ln1_b), tf1_Wq, r2d(tf1_bq), tf1_Wk, r2d(tf1_bk),
        tf1_Wv, r2d(tf1_bv), tf1_Wo, r2d(tf1_bo), r2d(tf1_ln2_g), r2d(tf1_ln2_b),
        tf1_W1, r2d(tf1_b1), tf1_W2, r2d(tf1_b2),
    ]

    in_specs = [
        pl.BlockSpec((1, S3, 1), lambda b: (b, 0, 0)),
        pl.BlockSpec((1, T - 1, S3), lambda b: (b, 0, 0)),
        pl.BlockSpec((1, 1, 1), lambda b: (b, 0, 0)),
    ] + [pl.BlockSpec(w.shape, _const_map(w.ndim)) for w in weights]

    out = pl.pallas_call(
        _body,
        grid=(B,),
        in_specs=in_specs,
        out_specs=pl.BlockSpec((1, 26, C), lambda b: (b, 0, 0)),
        out_shape=jax.ShapeDtypeStruct((B, 26, C), jnp.float32),
    )(t0col, actsf, ss.reshape(B, 1, 1), *weights)
    return out


# speedup vs baseline: 1253.6295x; 2.0847x over previous
import jax
import jax.numpy as jnp
from jax.experimental import pallas as pl

B, T, S, C = 8, 8, 8, 64
S3 = S ** 3
N = B * S3
L = T - 1
BL = B * L
NH, DH = 4, 16
N_TF = 2


def _layernorm(x, g, b, eps=1e-5):
    mu = jnp.mean(x, axis=-1, keepdims=True)
    var = jnp.mean((x - mu) ** 2, axis=-1, keepdims=True)
    return (x - mu) * jax.lax.rsqrt(var + eps) * g + b


def _body(t0_ref, acts_ref, ss_ref, *refs):
    out_ref = refs[-1]
    vals = [r[...] for r in refs[:-1]]
    (lin_in_W, lin_in_b,
     conv0_Wl, conv0_bl, conv0_Wr,
     conv1_Wl, conv1_bl, conv1_Wr,
     norm_g, norm_b,
     act_lin_W, act_lin_b,
     scalar_W, scalar_b) = vals[:14]
    tf = vals[14:]

    t0 = t0_ref[...]
    acts = acts_ref[...]
    ss = ss_ref[...]

    rowidx = jax.lax.broadcasted_iota(jnp.int32, (N, 1), 0)
    sid = rowidx % S3

    pr = jax.lax.broadcasted_iota(jnp.int32, (B, N), 0)
    pc = jax.lax.broadcasted_iota(jnp.int32, (B, N), 1)
    P = (pr == pc // S3).astype(jnp.float32)
    tr = jax.lax.broadcasted_iota(jnp.int32, (N, B), 0)
    tc = jax.lax.broadcasted_iota(jnp.int32, (N, B), 1)
    PT = (tr // S3 == tc).astype(jnp.float32)

    maskf_raw = (t0 != 0.0).astype(jnp.float32)
    cnt8 = jnp.dot(P, maskf_raw, preferred_element_type=jnp.float32)
    cnt_rows = jnp.dot(PT, cnt8, preferred_element_type=jnp.float32)
    maskf = jnp.where(jnp.logical_and(cnt_rows == 0.0, sid == 0),
                      1.0, maskf_raw)
    m_rows = jnp.maximum(cnt_rows, 1.0)
    deg = jnp.maximum(maskf * (m_rows - maskf), 1.0)
    inv_deg = 1.0 / deg

    inv_s1 = 1.0 / (S - 1)
    ii = (sid // (S * S)).astype(jnp.float32) * inv_s1
    jj = ((sid // S) % S).astype(jnp.float32) * inv_s1
    kk = (sid % S).astype(jnp.float32) * inv_s1
    ss_rows = jnp.dot(PT, ss, preferred_element_type=jnp.float32)

    x = (ii * lin_in_W[0:1, :] + jj * lin_in_W[1:2, :] + kk * lin_in_W[2:3, :]
         + (t0 * 0.5) * lin_in_W[3:4, :] + (ss_rows * (1.0 / S)) * lin_in_W[4:5, :]
         + lin_in_b)

    for Wl, bl, Wr in ((conv0_Wl, conv0_bl, conv0_Wr),
                       (conv1_Wl, conv1_bl, conv1_Wr)):
        Sx8 = jnp.dot(P, maskf * x, preferred_element_type=jnp.float32)
        SxRows = jnp.dot(PT, Sx8, preferred_element_type=jnp.float32)
        agg = maskf * (SxRows - x) * inv_deg
        h = (jnp.dot(agg, Wl, preferred_element_type=jnp.float32) + bl
             + jnp.dot(x, Wr, preferred_element_type=jnp.float32))
        x = _layernorm(jnp.maximum(h, 0.0), norm_g, norm_b)

    full = x * maskf

    r2 = jax.lax.broadcasted_iota(jnp.int32, (24, S3), 0)
    c2 = jax.lax.broadcasted_iota(jnp.int32, (24, S3), 1)
    sel = jnp.where(r2 < 8, c2 // (S * S), jnp.where(r2 < 16, (c2 // S) % S, c2 % S))
    poolW = (sel == (r2 % 8)).astype(jnp.float32)

    a = jnp.dot(acts, act_lin_W, preferred_element_type=jnp.float32) + act_lin_b
    br = jax.lax.broadcasted_iota(jnp.int32, (BL, BL), 0)
    bc = jax.lax.broadcasted_iota(jnp.int32, (BL, BL), 1)
    blockmask = (br // L == bc // L).astype(jnp.float32)
    inv_sqrt_dh = 1.0 / 4.0
    for l in range(N_TF):
        (ln1_g, ln1_b, Wq, bq, Wk, bk, Wv, bv, Wo, bo,
         ln2_g, ln2_b, W1, b1, W2, b2) = tf[l * 16:(l + 1) * 16]
        xa = _layernorm(a, ln1_g, ln1_b)
        q = jnp.dot(xa, Wq, preferred_element_type=jnp.float32) + bq
        k = jnp.dot(xa, Wk, preferred_element_type=jnp.float32) + bk
        v = jnp.dot(xa, Wv, preferred_element_type=jnp.float32) + bv
        outs = []
        for hh in range(NH):
            qh = q[:, hh * DH:(hh + 1) * DH]
            kh = k[:, hh * DH:(hh + 1) * DH]
            vh = v[:, hh * DH:(hh + 1) * DH]
            sc = jax.lax.dot_general(
                qh, kh, (((1,), (1,)), ((), ())),
                preferred_element_type=jnp.float32) * inv_sqrt_dh
            sc = sc - jnp.max(sc, axis=-1, keepdims=True)
            e = jnp.exp(sc) * blockmask
            att = e / jnp.sum(e, axis=-1, keepdims=True)
            outs.append(jnp.dot(att, vh, preferred_element_type=jnp.float32))
        o = jnp.concatenate(outs, axis=1)
        a = a + jnp.dot(o, Wo, preferred_element_type=jnp.float32) + bo
        h2 = _layernorm(a, ln2_g, ln2_b)
        ff = jnp.maximum(
            jnp.dot(h2, W1, preferred_element_type=jnp.float32) + b1, 0.0)
        a = a + jnp.dot(ff, W2, preferred_element_type=jnp.float32) + b2

    ar = jax.lax.broadcasted_iota(jnp.int32, (B, BL), 0)
    ac = jax.lax.broadcasted_iota(jnp.int32, (B, BL), 1)
    Pact = (ar == ac // L).astype(jnp.float32) * (1.0 / L)
    act_emb = jnp.dot(Pact, a, preferred_element_type=jnp.float32)

    mv_emb = jnp.maximum(ss * scalar_W + scalar_b, 0.0)

    for b in range(B):
        pooled = jnp.dot(poolW, full[b * S3:(b + 1) * S3, :],
                         preferred_element_type=jnp.float32) * (1.0 / 64.0)
        out_ref[b * 26:b * 26 + 24, :] = pooled
        out_ref[b * 26 + 24:b * 26 + 25, :] = act_emb[b:b + 1, :]
        out_ref[b * 26 + 25:b * 26 + 26, :] = mv_emb[b:b + 1, :]


def kernel(xx, ss, lin_in_W, lin_in_b, conv0_Wl, conv0_bl, conv0_Wr,
           conv1_Wl, conv1_bl, conv1_Wr, norm_g, norm_b,
           act_lin_W, act_lin_b, scalar_W, scalar_b,
           tf0_ln1_g, tf0_ln1_b, tf0_Wq, tf0_bq, tf0_Wk, tf0_bk,
           tf0_Wv, tf0_bv, tf0_Wo, tf0_bo, tf0_ln2_g, tf0_ln2_b,
           tf0_W1, tf0_b1, tf0_W2, tf0_b2,
           tf1_ln1_g, tf1_ln1_b, tf1_Wq, tf1_bq, tf1_Wk, tf1_bk,
           tf1_Wv, tf1_bv, tf1_Wo, tf1_bo, tf1_ln2_g, tf1_ln2_b,
           tf1_W1, tf1_b1, tf1_W2, tf1_b2):
    t0col = xx[:, 0].reshape(N, 1).astype(jnp.float32)
    actsf = xx[:, 1:].reshape(BL, S3).astype(jnp.float32)

    def r2d(v):
        return v.reshape(1, -1)

    weights = [
        lin_in_W, r2d(lin_in_b),
        conv0_Wl, r2d(conv0_bl), conv0_Wr,
        conv1_Wl, r2d(conv1_bl), conv1_Wr,
        r2d(norm_g), r2d(norm_b),
        act_lin_W, r2d(act_lin_b),
        scalar_W, r2d(scalar_b),
        r2d(tf0_ln1_g), r2d(tf0_ln1_b), tf0_Wq, r2d(tf0_bq), tf0_Wk, r2d(tf0_bk),
        tf0_Wv, r2d(tf0_bv), tf0_Wo, r2d(tf0_bo), r2d(tf0_ln2_g), r2d(tf0_ln2_b),
        tf0_W1, r2d(tf0_b1), tf0_W2, r2d(tf0_b2),
        r2d(tf1_ln1_g), r2d(tf1_ln1_b), tf1_Wq, r2d(tf1_bq), tf1_Wk, r2d(tf1_bk),
        tf1_Wv, r2d(tf1_bv), tf1_Wo, r2d(tf1_bo), r2d(tf1_ln2_g), r2d(tf1_ln2_b),
        tf1_W1, r2d(tf1_b1), tf1_W2, r2d(tf1_b2),
    ]

    out = pl.pallas_call(
        _body,
        out_shape=jax.ShapeDtypeStruct((B * 26, C), jnp.float32),
    )(t0col, actsf, ss, *weights)
    return out.reshape(B, 26, C)


# speedup vs baseline: 1330.7258x; 1.0615x over previous
import jax
import jax.numpy as jnp
from jax.experimental import pallas as pl

B, T, S, C = 8, 8, 8, 64
S3 = S ** 3
N = B * S3
L = T - 1
BL = B * L
NH, DH = 4, 16
N_TF = 2


def _layernorm(x, g, b, eps=1e-5):
    J = jnp.full((C, C), 1.0 / C, jnp.float32)
    mu = jnp.dot(x, J, preferred_element_type=jnp.float32)
    s2 = jnp.dot(x * x, J, preferred_element_type=jnp.float32)
    var = s2 - mu * mu
    return (x - mu) * jax.lax.rsqrt(var + eps) * g + b


def _body(t0_ref, acts_ref, ss_ref, *refs):
    out_ref = refs[-1]
    vals = [r[...] for r in refs[:-1]]
    (lin_in_W, lin_in_b,
     conv0_Wl, conv0_bl, conv0_Wr,
     conv1_Wl, conv1_bl, conv1_Wr,
     norm_g, norm_b,
     act_lin_W, act_lin_b,
     scalar_W, scalar_b) = vals[:14]
    tf = vals[14:]

    t0 = t0_ref[...]
    acts = acts_ref[...]
    ss = ss_ref[...]

    rowidx = jax.lax.broadcasted_iota(jnp.int32, (N, 1), 0)
    sid = rowidx % S3

    pr = jax.lax.broadcasted_iota(jnp.int32, (B, N), 0)
    pc = jax.lax.broadcasted_iota(jnp.int32, (B, N), 1)
    P = (pr == pc // S3).astype(jnp.float32)

    def bcast_rows(y8):
        k = y8.shape[1]
        return jnp.broadcast_to(y8[:, None, :], (B, S3, k)).reshape(N, k)

    maskf_raw = (t0 != 0.0).astype(jnp.float32)
    cnt8 = jnp.dot(P, maskf_raw, preferred_element_type=jnp.float32)
    cnt_rows = bcast_rows(cnt8)
    maskf = jnp.where(jnp.logical_and(cnt_rows == 0.0, sid == 0),
                      1.0, maskf_raw)
    m_rows = jnp.maximum(cnt_rows, 1.0)
    deg = jnp.maximum(maskf * (m_rows - maskf), 1.0)
    md = maskf / deg

    inv_s1 = 1.0 / (S - 1)
    ii = (sid // (S * S)).astype(jnp.float32) * inv_s1
    jj = ((sid // S) % S).astype(jnp.float32) * inv_s1
    kk = (sid % S).astype(jnp.float32) * inv_s1
    ss_rows = bcast_rows(ss)

    x = (ii * lin_in_W[0:1, :] + jj * lin_in_W[1:2, :] + kk * lin_in_W[2:3, :]
         + (t0 * 0.5) * lin_in_W[3:4, :] + (ss_rows * (1.0 / S)) * lin_in_W[4:5, :]
         + lin_in_b)

    for Wl, bl, Wr in ((conv0_Wl, conv0_bl, conv0_Wr),
                       (conv1_Wl, conv1_bl, conv1_Wr)):
        Sx8 = jnp.dot(P, maskf * x, preferred_element_type=jnp.float32)
        SxRows = bcast_rows(Sx8)
        agg = md * (SxRows - x)
        h = (jnp.dot(agg, Wl, preferred_element_type=jnp.float32) + bl
             + jnp.dot(x, Wr, preferred_element_type=jnp.float32))
        x = _layernorm(jnp.maximum(h, 0.0), norm_g, norm_b)

    full = x * maskf

    r2 = jax.lax.broadcasted_iota(jnp.int32, (24, S3), 0)
    c2 = jax.lax.broadcasted_iota(jnp.int32, (24, S3), 1)
    sel = jnp.where(r2 < 8, c2 // (S * S), jnp.where(r2 < 16, (c2 // S) % S, c2 % S))
    poolW = (sel == (r2 % 8)).astype(jnp.float32)

    a = jnp.dot(acts, act_lin_W, preferred_element_type=jnp.float32) + act_lin_b
    br = jax.lax.broadcasted_iota(jnp.int32, (BL, BL), 0)
    bc = jax.lax.broadcasted_iota(jnp.int32, (BL, BL), 1)
    blockmask = (br // L == bc // L).astype(jnp.float32)
    inv_sqrt_dh = 1.0 / 4.0
    for l in range(N_TF):
        (ln1_g, ln1_b, Wq, bq, Wk, bk, Wv, bv, Wo, bo,
         ln2_g, ln2_b, W1, b1, W2, b2) = tf[l * 16:(l + 1) * 16]
        xa = _layernorm(a, ln1_g, ln1_b)
        q = jnp.dot(xa, Wq, preferred_element_type=jnp.float32) + bq
        k = jnp.dot(xa, Wk, preferred_element_type=jnp.float32) + bk
        v = jnp.dot(xa, Wv, preferred_element_type=jnp.float32) + bv
        outs = []
        for hh in range(NH):
            qh = q[:, hh * DH:(hh + 1) * DH]
            kh = k[:, hh * DH:(hh + 1) * DH]
            vh = v[:, hh * DH:(hh + 1) * DH]
            sc = jax.lax.dot_general(
                qh, kh, (((1,), (1,)), ((), ())),
                preferred_element_type=jnp.float32) * inv_sqrt_dh
            sc = sc - jnp.max(sc, axis=-1, keepdims=True)
            e = jnp.exp(sc) * blockmask
            att = e / jnp.sum(e, axis=-1, keepdims=True)
            outs.append(jnp.dot(att, vh, preferred_element_type=jnp.float32))
        o = jnp.concatenate(outs, axis=1)
        a = a + jnp.dot(o, Wo, preferred_element_type=jnp.float32) + bo
        h2 = _layernorm(a, ln2_g, ln2_b)
        ff = jnp.maximum(
            jnp.dot(h2, W1, preferred_element_type=jnp.float32) + b1, 0.0)
        a = a + jnp.dot(ff, W2, preferred_element_type=jnp.float32) + b2

    ar = jax.lax.broadcasted_iota(jnp.int32, (B, BL), 0)
    ac = jax.lax.broadcasted_iota(jnp.int32, (B, BL), 1)
    Pact = (ar == ac // L).astype(jnp.float32) * (1.0 / L)
    act_emb = jnp.dot(Pact, a, preferred_element_type=jnp.float32)

    mv_emb = jnp.maximum(ss * scalar_W + scalar_b, 0.0)

    for b in range(B):
        pooled = jnp.dot(poolW, full[b * S3:(b + 1) * S3, :],
                         preferred_element_type=jnp.float32) * (1.0 / 64.0)
        out_ref[b * 26:b * 26 + 24, :] = pooled
        out_ref[b * 26 + 24:b * 26 + 25, :] = act_emb[b:b + 1, :]
        out_ref[b * 26 + 25:b * 26 + 26, :] = mv_emb[b:b + 1, :]


def kernel(xx, ss, lin_in_W, lin_in_b, conv0_Wl, conv0_bl, conv0_Wr,
           conv1_Wl, conv1_bl, conv1_Wr, norm_g, norm_b,
           act_lin_W, act_lin_b, scalar_W, scalar_b,
           tf0_ln1_g, tf0_ln1_b, tf0_Wq, tf0_bq, tf0_Wk, tf0_bk,
           tf0_Wv, tf0_bv, tf0_Wo, tf0_bo, tf0_ln2_g, tf0_ln2_b,
           tf0_W1, tf0_b1, tf0_W2, tf0_b2,
           tf1_ln1_g, tf1_ln1_b, tf1_Wq, tf1_bq, tf1_Wk, tf1_bk,
           tf1_Wv, tf1_bv, tf1_Wo, tf1_bo, tf1_ln2_g, tf1_ln2_b,
           tf1_W1, tf1_b1, tf1_W2, tf1_b2):
    t0col = xx[:, 0].reshape(N, 1).astype(jnp.float32)
    actsf = xx[:, 1:].reshape(BL, S3).astype(jnp.float32)

    def r2d(v):
        return v.reshape(1, -1)

    weights = [
        lin_in_W, r2d(lin_in_b),
        conv0_Wl, r2d(conv0_bl), conv0_Wr,
        conv1_Wl, r2d(conv1_bl), conv1_Wr,
        r2d(norm_g), r2d(norm_b),
        act_lin_W, r2d(act_lin_b),
        scalar_W, r2d(scalar_b),
        r2d(tf0_ln1_g), r2d(tf0_ln1_b), tf0_Wq, r2d(tf0_bq), tf0_Wk, r2d(tf0_bk),
        tf0_Wv, r2d(tf0_bv), tf0_Wo, r2d(tf0_bo), r2d(tf0_ln2_g), r2d(tf0_ln2_b),
        tf0_W1, r2d(tf0_b1), tf0_W2, r2d(tf0_b2),
        r2d(tf1_ln1_g), r2d(tf1_ln1_b), tf1_Wq, r2d(tf1_bq), tf1_Wk, r2d(tf1_bk),
        tf1_Wv, r2d(tf1_bv), tf1_Wo, r2d(tf1_bo), r2d(tf1_ln2_g), r2d(tf1_ln2_b),
        tf1_W1, r2d(tf1_b1), tf1_W2, r2d(tf1_b2),
    ]

    out = pl.pallas_call(
        _body,
        out_shape=jax.ShapeDtypeStruct((B * 26, C), jnp.float32),
    )(t0col, actsf, ss, *weights)
    return out.reshape(B, 26, C)
